# baseline (speedup 1.0000x reference)
import functools
import math

import numpy as np
import jax
import jax.numpy as jnp
from jax import lax
from jax.experimental import pallas as pl
from jax.experimental.pallas import tpu as pltpu
from jax.experimental.pallas import tpu_sc as plsc

_N = 10000
_D_IN = 128
_FEAT = 16
_E = 320000
_ACT = 4096
_HID = 256
_T = 3
_TEMB = 16
_MAX_ACTION = 20.0

_NPAD = 10240
_NW = 32
_ECHUNK = 1000
_EPW_ROWS = 10
_SLICE = _NPAD // 16

_i = np.arange(1, _T + 1, dtype=np.float64)
_BETAS = 1.0 - np.exp(-0.1 / _T - 0.5 * (10.0 - 0.1) * (2.0 * _i - 1.0) / (_T ** 2))
_ALPHAS = 1.0 - _BETAS
_ACP = np.cumprod(_ALPHAS)

_SB = np.sqrt(1.0 - _ACP)
_ISA = 1.0 / np.sqrt(_ACP)
_ACP_PREV = np.concatenate([[1.0], _ACP[:-1]])
_C1 = _BETAS * np.sqrt(_ACP_PREV) / (1.0 - _ACP)
_C2 = (1.0 - _ACP_PREV) * np.sqrt(_ALPHAS) / (1.0 - _ACP)
_SIG = np.sqrt(_BETAS * (1.0 - _ACP_PREV) / (1.0 - _ACP))

_half = _TEMB // 2
_freqs = np.exp(-math.log(10000.0) * np.arange(_half, dtype=np.float64) / _half)
_TE = np.concatenate(
    [np.sin(np.arange(_T)[:, None] * _freqs[None, :]),
     np.cos(np.arange(_T)[:, None] * _freqs[None, :])], axis=1
).astype(np.float32)

def _sc_deg_body(edge_hbm, out_hbm, idx_v, ones_v, zb_v, shared_deg, isem):
    cid = lax.axis_index("c")
    sid = lax.axis_index("s")
    wid = cid * 16 + sid

    idescs = [
        pltpu.async_copy(
            edge_hbm.at[1, pl.ds((wid * _EPW_ROWS + j) * _ECHUNK, _ECHUNK)],
            idx_v.at[j], isem)
        for j in range(_EPW_ROWS)
    ]

    def z16(i, carry):
        zb_v[pl.ds(i * 16, 16)] = jnp.zeros((16,), jnp.float32)
        return carry
    lax.fori_loop(0, _SLICE // 16, z16, 0)

    def o16(i, carry):
        ones_v[pl.ds(i * 16, 16)] = jnp.full((16,), 1.0, jnp.float32)
        return carry
    lax.fori_loop(0, _ECHUNK // 16, o16, 0)
    ones_v[pl.ds(_ECHUNK - 16, 16)] = jnp.full((16,), 1.0, jnp.float32)

    pltpu.sync_copy(zb_v, shared_deg.at[pl.ds(sid * _SLICE, _SLICE)])
    for d in idescs:
        d.wait()
    plsc.subcore_barrier()

    def chunk(j, carry):
        pltpu.sync_copy(ones_v, shared_deg.at[idx_v.at[j]], add=True)
        return carry
    lax.fori_loop(0, _EPW_ROWS, chunk, 0)
    plsc.subcore_barrier()
    pltpu.sync_copy(shared_deg.at[pl.ds(sid * _SLICE, _SLICE)],
                    out_hbm.at[cid, pl.ds(sid * _SLICE, _SLICE)])


@functools.lru_cache(maxsize=None)
def _get_deg_call():
    mesh = plsc.VectorSubcoreMesh(
        core_axis_name="c", subcore_axis_name="s", num_cores=2, num_subcores=16)
    return pl.kernel(
        _sc_deg_body,
        out_type=jax.ShapeDtypeStruct((2, _NPAD), jnp.float32),
        mesh=mesh,
        compiler_params=pltpu.CompilerParams(use_tc_tiling_on_sc=False),
        scratch_types=[
            pltpu.VMEM((_EPW_ROWS, _ECHUNK), jnp.int32),
            pltpu.VMEM((_ECHUNK,), jnp.float32),
            pltpu.VMEM((_SLICE,), jnp.float32),
            pltpu.VMEM_SHARED((_NPAD,), jnp.float32),
            pltpu.SemaphoreType.DMA,
        ],
    )


def _sc_msg_body(edge_hbm, proj_hbm, out_hbm,
                 sidx_v, didx_v, rows0, rows1, zb_v, shared_agg,
                 isem, gsem0, gsem1):
    cid = lax.axis_index("c")
    sid = lax.axis_index("s")
    wid = cid * 16 + sid

    idescs = [
        pltpu.async_copy(
            edge_hbm.at[e, pl.ds((wid * _EPW_ROWS + j) * _ECHUNK, _ECHUNK)],
            (sidx_v if e == 0 else didx_v).at[j], isem)
        for e in (0, 1) for j in range(_EPW_ROWS)
    ]

    def z1(i, carry):
        zb_v[i] = jnp.zeros((16,), jnp.float32)
        return carry
    lax.fori_loop(0, _SLICE, z1, 0)

    pltpu.sync_copy(zb_v, shared_agg.at[pl.ds(sid * _SLICE, _SLICE)])
    for d in idescs:
        d.wait()
    plsc.subcore_barrier()

    rows = (rows0, rows1)
    gsems = (gsem0, gsem1)
    descs = [None, None]
    descs[0] = pltpu.async_copy(proj_hbm.at[sidx_v.at[0]], rows0, gsem0)
    for j in range(_EPW_ROWS):
        b = j % 2
        descs[b].wait()
        if j + 1 < _EPW_ROWS:
            nb = (j + 1) % 2
            descs[nb] = pltpu.async_copy(
                proj_hbm.at[sidx_v.at[j + 1]], rows[nb], gsems[nb])
        pltpu.sync_copy(rows[b], shared_agg.at[didx_v.at[j]], add=True)
    plsc.subcore_barrier()
    pltpu.sync_copy(shared_agg.at[pl.ds(sid * _SLICE, _SLICE)],
                    out_hbm.at[cid, pl.ds(sid * _SLICE, _SLICE)])


@functools.lru_cache(maxsize=None)
def _get_msg_call():
    mesh = plsc.VectorSubcoreMesh(
        core_axis_name="c", subcore_axis_name="s", num_cores=2, num_subcores=16)
    return pl.kernel(
        _sc_msg_body,
        out_type=jax.ShapeDtypeStruct((2, _NPAD, _FEAT), jnp.float32),
        mesh=mesh,
        compiler_params=pltpu.CompilerParams(use_tc_tiling_on_sc=False),
        scratch_types=[
            pltpu.VMEM((_EPW_ROWS, _ECHUNK), jnp.int32),
            pltpu.VMEM((_EPW_ROWS, _ECHUNK), jnp.int32),
            pltpu.VMEM((_ECHUNK, _FEAT), jnp.float32),
            pltpu.VMEM((_ECHUNK, _FEAT), jnp.float32),
            pltpu.VMEM((_SLICE, _FEAT), jnp.float32),
            pltpu.VMEM_SHARED((_NPAD, _FEAT), jnp.float32),
            pltpu.SemaphoreType.DMA,
            pltpu.SemaphoreType.DMA,
            pltpu.SemaphoreType.DMA,
        ],
    )


_RB = 2048


def _proj_body(deg_ref, f_ref, wg_ref, out_ref):
    dv = deg_ref[...]
    isd = lax.rsqrt(jnp.maximum(dv[0:1, :_N] + dv[1:2, :_N], 1.0))
    isd_c = jnp.transpose(isd, (1, 0))
    p = jnp.dot(f_ref[...], wg_ref[...], preferred_element_type=jnp.float32)
    out_ref[...] = p * isd_c


_proj_call = pl.pallas_call(
    _proj_body,
    in_specs=[
        pl.BlockSpec(memory_space=pltpu.VMEM),
        pl.BlockSpec(memory_space=pltpu.VMEM),
        pl.BlockSpec(memory_space=pltpu.VMEM),
    ],
    out_specs=pl.BlockSpec(memory_space=pltpu.VMEM),
    out_shape=jax.ShapeDtypeStruct((_N, _FEAT), jnp.float32),
)


def _emb_body(agg_ref, deg_ref, bg_ref, out_ref):
    av = agg_ref[...]
    s = av[0, :_N] + av[1, :_N]
    dv = deg_ref[...]
    isd = lax.rsqrt(jnp.maximum(dv[0:1, :_N] + dv[1:2, :_N], 1.0))
    isd_c = jnp.transpose(isd, (1, 0))
    out_ref[...] = jnp.maximum(s * isd_c + bg_ref[...], 0.0)


_emb_call = pl.pallas_call(
    _emb_body,
    in_specs=[
        pl.BlockSpec(memory_space=pltpu.VMEM),
        pl.BlockSpec(memory_space=pltpu.VMEM),
        pl.BlockSpec(memory_space=pltpu.VMEM),
    ],
    out_specs=pl.BlockSpec(memory_space=pltpu.VMEM),
    out_shape=jax.ShapeDtypeStruct((_N, _FEAT), jnp.float32),
)


_KB = 5000
_NKB = (_N * _FEAT) // _KB
_NBUF = 4
_W1S_OFF = _ACT + _TEMB


def _head_body(state_ref, b1_ref, w2_ref, b2_ref, w3_ref, b3_ref, te_ref,
               xi_ref, nz_ref, w1_hbm, out_ref, wb0, wb1, wb2, wb3, wbm,
               sem0, sem1, sem2, sem3, semm):
    cm = pltpu.make_async_copy(w1_hbm.at[pl.ds(0, _W1S_OFF), :], wbm, semm)
    cm.start()
    wbs = (wb0, wb1, wb2, wb3)
    sems = (sem0, sem1, sem2, sem3)

    def cp(k):
        b = k % _NBUF
        return pltpu.make_async_copy(
            w1_hbm.at[pl.ds(_W1S_OFF + k * _KB, _KB), :], wbs[b], sems[b])

    for k in range(_NBUF - 1):
        cp(k).start()
    acc = jnp.zeros((1, _HID), jnp.float32)
    for k in range(_NKB):
        if k + _NBUF - 1 < _NKB:
            cp(k + _NBUF - 1).start()
        cp(k).wait()
        acc = acc + jnp.dot(state_ref[:, pl.ds(k * _KB, _KB)],
                            wbs[k % _NBUF][...],
                            preferred_element_type=jnp.float32)
    cm.wait()
    wb = wbm
    base = acc + b1_ref[...]
    te_v = te_ref[...]
    x = xi_ref[...]
    w1te = wb[pl.ds(_ACT, _TEMB), :]
    for t in (2, 1, 0):
        tp = jnp.dot(te_v[t:t + 1, :], w1te, preferred_element_type=jnp.float32)
        h = jnp.dot(x, wb[pl.ds(0, _ACT), :], preferred_element_type=jnp.float32)
        h = jnp.maximum(h + tp + base, 0.0)
        h = jnp.maximum(
            jnp.dot(h, w2_ref[...], preferred_element_type=jnp.float32)
            + b2_ref[...], 0.0)
        eps = jnp.dot(h, w3_ref[...], preferred_element_type=jnp.float32) + b3_ref[...]
        x0 = jnp.clip((x - float(_SB[t]) * eps) * float(_ISA[t]),
                      -_MAX_ACTION, _MAX_ACTION)
        x = float(_C1[t]) * x0 + float(_C2[t]) * x
        if t > 0:
            x = x + float(_SIG[t]) * nz_ref[2 - t:3 - t, :]
    out_ref[...] = x


_head_call = pl.pallas_call(
    _head_body,
    in_specs=[
        pl.BlockSpec(memory_space=pltpu.VMEM),
        pl.BlockSpec(memory_space=pltpu.VMEM),
        pl.BlockSpec(memory_space=pltpu.VMEM),
        pl.BlockSpec(memory_space=pltpu.VMEM),
        pl.BlockSpec(memory_space=pltpu.VMEM),
        pl.BlockSpec(memory_space=pltpu.VMEM),
        pl.BlockSpec(memory_space=pltpu.VMEM),
        pl.BlockSpec(memory_space=pltpu.VMEM),
        pl.BlockSpec(memory_space=pltpu.VMEM),
        pl.BlockSpec(memory_space=pl.ANY),
    ],
    out_specs=pl.BlockSpec(memory_space=pltpu.VMEM),
    out_shape=jax.ShapeDtypeStruct((1, _ACT), jnp.float32),
    scratch_shapes=[
        pltpu.VMEM((_KB, _HID), jnp.float32),
        pltpu.VMEM((_KB, _HID), jnp.float32),
        pltpu.VMEM((_KB, _HID), jnp.float32),
        pltpu.VMEM((_KB, _HID), jnp.float32),
        pltpu.VMEM((_W1S_OFF, _HID), jnp.float32),
        pltpu.SemaphoreType.DMA,
        pltpu.SemaphoreType.DMA,
        pltpu.SemaphoreType.DMA,
        pltpu.SemaphoreType.DMA,
        pltpu.SemaphoreType.DMA,
    ],
)


def kernel(feature_matrix, edge_index, Wg, bg, W1, b1, W2, b2, W3, b3):
    deg_p = _get_deg_call()(edge_index)
    proj = _proj_call(deg_p, feature_matrix, Wg)
    agg_p = _get_msg_call()(edge_index, proj)
    emb = _emb_call(agg_p, deg_p, bg.reshape(1, _FEAT))
    state = emb.reshape(1, _N * _FEAT)

    xi = jax.random.normal(jax.random.key(42), (1, _ACT), dtype=jnp.float32)
    n2 = jax.random.normal(jax.random.fold_in(jax.random.key(7), 2),
                           (1, _ACT), dtype=jnp.float32)
    n1 = jax.random.normal(jax.random.fold_in(jax.random.key(7), 1),
                           (1, _ACT), dtype=jnp.float32)
    noise = jnp.concatenate([n2, n1], axis=0)
    te = jnp.asarray(_TE)

    logits = _head_call(state, b1.reshape(1, _HID), W2, b2.reshape(1, _HID),
                        W3, b3.reshape(1, _ACT), te, xi, noise, W1)
    return logits


# speedup vs baseline: 1.0119x; 1.0119x over previous
import functools
import math

import numpy as np
import jax
import jax.numpy as jnp
from jax import lax
from jax.experimental import pallas as pl
from jax.experimental.pallas import tpu as pltpu
from jax.experimental.pallas import tpu_sc as plsc

_N = 10000
_D_IN = 128
_FEAT = 16
_E = 320000
_ACT = 4096
_HID = 256
_T = 3
_TEMB = 16
_MAX_ACTION = 20.0

_NPAD = 10240
_NW = 32
_ECHUNK = 1000
_EPW_ROWS = 10
_SLICE = _NPAD // 16

_i = np.arange(1, _T + 1, dtype=np.float64)
_BETAS = 1.0 - np.exp(-0.1 / _T - 0.5 * (10.0 - 0.1) * (2.0 * _i - 1.0) / (_T ** 2))
_ALPHAS = 1.0 - _BETAS
_ACP = np.cumprod(_ALPHAS)

_SB = np.sqrt(1.0 - _ACP)
_ISA = 1.0 / np.sqrt(_ACP)
_ACP_PREV = np.concatenate([[1.0], _ACP[:-1]])
_C1 = _BETAS * np.sqrt(_ACP_PREV) / (1.0 - _ACP)
_C2 = (1.0 - _ACP_PREV) * np.sqrt(_ALPHAS) / (1.0 - _ACP)
_SIG = np.sqrt(_BETAS * (1.0 - _ACP_PREV) / (1.0 - _ACP))

_half = _TEMB // 2
_freqs = np.exp(-math.log(10000.0) * np.arange(_half, dtype=np.float64) / _half)
_TE = np.concatenate(
    [np.sin(np.arange(_T)[:, None] * _freqs[None, :]),
     np.cos(np.arange(_T)[:, None] * _freqs[None, :])], axis=1
).astype(np.float32)

def _sc_deg_body(edge_hbm, out_hbm, idx_v, ones_v, zb_v, shared_deg, isem):
    cid = lax.axis_index("c")
    sid = lax.axis_index("s")
    wid = cid * 16 + sid

    idescs = [
        pltpu.async_copy(
            edge_hbm.at[1, pl.ds((wid * _EPW_ROWS + j) * _ECHUNK, _ECHUNK)],
            idx_v.at[j], isem)
        for j in range(_EPW_ROWS)
    ]

    def z16(i, carry):
        zb_v[pl.ds(i * 16, 16)] = jnp.zeros((16,), jnp.float32)
        return carry
    lax.fori_loop(0, _SLICE // 16, z16, 0)

    def o16(i, carry):
        ones_v[pl.ds(i * 16, 16)] = jnp.full((16,), 1.0, jnp.float32)
        return carry
    lax.fori_loop(0, _ECHUNK // 16, o16, 0)
    ones_v[pl.ds(_ECHUNK - 16, 16)] = jnp.full((16,), 1.0, jnp.float32)

    pltpu.sync_copy(zb_v, shared_deg.at[pl.ds(sid * _SLICE, _SLICE)])
    for d in idescs:
        d.wait()
    plsc.subcore_barrier()

    def chunk(j, carry):
        pltpu.sync_copy(ones_v, shared_deg.at[idx_v.at[j]], add=True)
        return carry
    lax.fori_loop(0, _EPW_ROWS, chunk, 0)
    plsc.subcore_barrier()
    pltpu.sync_copy(shared_deg.at[pl.ds(sid * _SLICE, _SLICE)],
                    out_hbm.at[cid, pl.ds(sid * _SLICE, _SLICE)])


@functools.lru_cache(maxsize=None)
def _get_deg_call():
    mesh = plsc.VectorSubcoreMesh(
        core_axis_name="c", subcore_axis_name="s", num_cores=2, num_subcores=16)
    return pl.kernel(
        _sc_deg_body,
        out_type=jax.ShapeDtypeStruct((2, _NPAD), jnp.float32),
        mesh=mesh,
        compiler_params=pltpu.CompilerParams(use_tc_tiling_on_sc=False),
        scratch_types=[
            pltpu.VMEM((_EPW_ROWS, _ECHUNK), jnp.int32),
            pltpu.VMEM((_ECHUNK,), jnp.float32),
            pltpu.VMEM((_SLICE,), jnp.float32),
            pltpu.VMEM_SHARED((_NPAD,), jnp.float32),
            pltpu.SemaphoreType.DMA,
        ],
    )


def _sc_msg_body(edge_hbm, proj_hbm, out_hbm,
                 sidx_v, didx_v, rows0, rows1, zb_v, shared_agg,
                 isem, gsem0, gsem1):
    cid = lax.axis_index("c")
    sid = lax.axis_index("s")
    wid = cid * 16 + sid

    idescs = [
        pltpu.async_copy(
            edge_hbm.at[e, pl.ds((wid * _EPW_ROWS + j) * _ECHUNK, _ECHUNK)],
            (sidx_v if e == 0 else didx_v).at[j], isem)
        for e in (0, 1) for j in range(_EPW_ROWS)
    ]

    def z1(i, carry):
        zb_v[i] = jnp.zeros((16,), jnp.float32)
        return carry
    lax.fori_loop(0, _SLICE, z1, 0)

    pltpu.sync_copy(zb_v, shared_agg.at[pl.ds(sid * _SLICE, _SLICE)])
    for d in idescs:
        d.wait()
    plsc.subcore_barrier()

    rows = (rows0, rows1)
    gsems = (gsem0, gsem1)
    descs = [None, None]
    descs[0] = pltpu.async_copy(proj_hbm.at[sidx_v.at[0]], rows0, gsem0)
    for j in range(_EPW_ROWS):
        b = j % 2
        descs[b].wait()
        if j + 1 < _EPW_ROWS:
            nb = (j + 1) % 2
            descs[nb] = pltpu.async_copy(
                proj_hbm.at[sidx_v.at[j + 1]], rows[nb], gsems[nb])
        pltpu.sync_copy(rows[b], shared_agg.at[didx_v.at[j]], add=True)
    plsc.subcore_barrier()
    pltpu.sync_copy(shared_agg.at[pl.ds(sid * _SLICE, _SLICE)],
                    out_hbm.at[cid, pl.ds(sid * _SLICE, _SLICE)])


@functools.lru_cache(maxsize=None)
def _get_msg_call():
    mesh = plsc.VectorSubcoreMesh(
        core_axis_name="c", subcore_axis_name="s", num_cores=2, num_subcores=16)
    return pl.kernel(
        _sc_msg_body,
        out_type=jax.ShapeDtypeStruct((2, _NPAD, _FEAT), jnp.float32),
        mesh=mesh,
        compiler_params=pltpu.CompilerParams(use_tc_tiling_on_sc=False),
        scratch_types=[
            pltpu.VMEM((_EPW_ROWS, _ECHUNK), jnp.int32),
            pltpu.VMEM((_EPW_ROWS, _ECHUNK), jnp.int32),
            pltpu.VMEM((_ECHUNK, _FEAT), jnp.float32),
            pltpu.VMEM((_ECHUNK, _FEAT), jnp.float32),
            pltpu.VMEM((_SLICE, _FEAT), jnp.float32),
            pltpu.VMEM_SHARED((_NPAD, _FEAT), jnp.float32),
            pltpu.SemaphoreType.DMA,
            pltpu.SemaphoreType.DMA,
            pltpu.SemaphoreType.DMA,
        ],
    )


_RB = 2048


def _praw_body(f_ref, wg_ref, out_ref):
    out_ref[...] = jnp.dot(f_ref[...], wg_ref[...],
                           preferred_element_type=jnp.float32)


_praw_call = pl.pallas_call(
    _praw_body,
    in_specs=[
        pl.BlockSpec(memory_space=pltpu.VMEM),
        pl.BlockSpec(memory_space=pltpu.VMEM),
    ],
    out_specs=pl.BlockSpec(memory_space=pltpu.VMEM),
    out_shape=jax.ShapeDtypeStruct((_N, _FEAT), jnp.float32),
)


def _pscale_body(deg_ref, p_ref, out_ref):
    dv = deg_ref[...]
    isd = lax.rsqrt(jnp.maximum(dv[0:1, :_N] + dv[1:2, :_N], 1.0))
    isd_c = jnp.transpose(isd, (1, 0))
    out_ref[...] = p_ref[...] * isd_c


_pscale_call = pl.pallas_call(
    _pscale_body,
    in_specs=[
        pl.BlockSpec(memory_space=pltpu.VMEM),
        pl.BlockSpec(memory_space=pltpu.VMEM),
    ],
    out_specs=pl.BlockSpec(memory_space=pltpu.VMEM),
    out_shape=jax.ShapeDtypeStruct((_N, _FEAT), jnp.float32),
)


def _emb_body(agg_ref, deg_ref, bg_ref, out_ref):
    av = agg_ref[...]
    s = av[0, :_N] + av[1, :_N]
    dv = deg_ref[...]
    isd = lax.rsqrt(jnp.maximum(dv[0:1, :_N] + dv[1:2, :_N], 1.0))
    isd_c = jnp.transpose(isd, (1, 0))
    out_ref[...] = jnp.maximum(s * isd_c + bg_ref[...], 0.0)


_emb_call = pl.pallas_call(
    _emb_body,
    in_specs=[
        pl.BlockSpec(memory_space=pltpu.VMEM),
        pl.BlockSpec(memory_space=pltpu.VMEM),
        pl.BlockSpec(memory_space=pltpu.VMEM),
    ],
    out_specs=pl.BlockSpec(memory_space=pltpu.VMEM),
    out_shape=jax.ShapeDtypeStruct((_N, _FEAT), jnp.float32),
)


_KB = 8000
_NKB = (_N * _FEAT) // _KB
_W1S_OFF = _ACT + _TEMB


def _head_body(state_ref, b1_ref, w2_ref, b2_ref, w3_ref, b3_ref, te_ref,
               xi_ref, nz_ref, w1_hbm, out_ref, wb0, wb1, wbm,
               sem0, sem1, semm):
    cm = pltpu.make_async_copy(w1_hbm.at[pl.ds(0, _W1S_OFF), :], wbm, semm)
    cm.start()
    wbs = (wb0, wb1)
    sems = (sem0, sem1)

    def cp(k, b):
        return pltpu.make_async_copy(
            w1_hbm.at[pl.ds(_W1S_OFF + k * _KB, _KB), :], wbs[b], sems[b])

    cp(0, 0).start()
    acc = jnp.zeros((1, _HID), jnp.float32)
    for k in range(_NKB):
        if k + 1 < _NKB:
            cp(k + 1, (k + 1) % 2).start()
        cp(k, k % 2).wait()
        acc = acc + jnp.dot(state_ref[:, pl.ds(k * _KB, _KB)],
                            wbs[k % 2][...],
                            preferred_element_type=jnp.float32)
    cm.wait()
    wb = wbm
    base = acc + b1_ref[...]
    te_v = te_ref[...]
    x = xi_ref[...]
    w1te = wb[pl.ds(_ACT, _TEMB), :]
    for t in (2, 1, 0):
        tp = jnp.dot(te_v[t:t + 1, :], w1te, preferred_element_type=jnp.float32)
        h = jnp.dot(x, wb[pl.ds(0, _ACT), :], preferred_element_type=jnp.float32)
        h = jnp.maximum(h + tp + base, 0.0)
        h = jnp.maximum(
            jnp.dot(h, w2_ref[...], preferred_element_type=jnp.float32)
            + b2_ref[...], 0.0)
        eps = jnp.dot(h, w3_ref[...], preferred_element_type=jnp.float32) + b3_ref[...]
        x0 = jnp.clip((x - float(_SB[t]) * eps) * float(_ISA[t]),
                      -_MAX_ACTION, _MAX_ACTION)
        x = float(_C1[t]) * x0 + float(_C2[t]) * x
        if t > 0:
            x = x + float(_SIG[t]) * nz_ref[2 - t:3 - t, :]
    out_ref[...] = x


_head_call = pl.pallas_call(
    _head_body,
    in_specs=[
        pl.BlockSpec(memory_space=pltpu.VMEM),
        pl.BlockSpec(memory_space=pltpu.VMEM),
        pl.BlockSpec(memory_space=pltpu.VMEM),
        pl.BlockSpec(memory_space=pltpu.VMEM),
        pl.BlockSpec(memory_space=pltpu.VMEM),
        pl.BlockSpec(memory_space=pltpu.VMEM),
        pl.BlockSpec(memory_space=pltpu.VMEM),
        pl.BlockSpec(memory_space=pltpu.VMEM),
        pl.BlockSpec(memory_space=pltpu.VMEM),
        pl.BlockSpec(memory_space=pl.ANY),
    ],
    out_specs=pl.BlockSpec(memory_space=pltpu.VMEM),
    out_shape=jax.ShapeDtypeStruct((1, _ACT), jnp.float32),
    scratch_shapes=[
        pltpu.VMEM((_KB, _HID), jnp.float32),
        pltpu.VMEM((_KB, _HID), jnp.float32),
        pltpu.VMEM((_W1S_OFF, _HID), jnp.float32),
        pltpu.SemaphoreType.DMA,
        pltpu.SemaphoreType.DMA,
        pltpu.SemaphoreType.DMA,
    ],
)


def kernel(feature_matrix, edge_index, Wg, bg, W1, b1, W2, b2, W3, b3):
    praw = _praw_call(feature_matrix, Wg)
    deg_p = _get_deg_call()(edge_index)
    proj = _pscale_call(deg_p, praw)
    agg_p = _get_msg_call()(edge_index, proj)
    emb = _emb_call(agg_p, deg_p, bg.reshape(1, _FEAT))
    state = emb.reshape(1, _N * _FEAT)

    xi = jax.random.normal(jax.random.key(42), (1, _ACT), dtype=jnp.float32)
    n2 = jax.random.normal(jax.random.fold_in(jax.random.key(7), 2),
                           (1, _ACT), dtype=jnp.float32)
    n1 = jax.random.normal(jax.random.fold_in(jax.random.key(7), 1),
                           (1, _ACT), dtype=jnp.float32)
    noise = jnp.concatenate([n2, n1], axis=0)
    te = jnp.asarray(_TE)

    logits = _head_call(state, b1.reshape(1, _HID), W2, b2.reshape(1, _HID),
                        W3, b3.reshape(1, _ACT), te, xi, noise, W1)
    return logits


# speedup vs baseline: 1.0334x; 1.0212x over previous
import functools
import math

import numpy as np
import jax
import jax.numpy as jnp
from jax import lax
from jax.experimental import pallas as pl
from jax.experimental.pallas import tpu as pltpu
from jax.experimental.pallas import tpu_sc as plsc

_N = 10000
_D_IN = 128
_FEAT = 16
_E = 320000
_ACT = 4096
_HID = 256
_T = 3
_TEMB = 16
_MAX_ACTION = 20.0

_NPAD = 10240
_NW = 32
_ECHUNK = 2000
_EPW_ROWS = 5
_SLICE = _NPAD // 16

_i = np.arange(1, _T + 1, dtype=np.float64)
_BETAS = 1.0 - np.exp(-0.1 / _T - 0.5 * (10.0 - 0.1) * (2.0 * _i - 1.0) / (_T ** 2))
_ALPHAS = 1.0 - _BETAS
_ACP = np.cumprod(_ALPHAS)

_SB = np.sqrt(1.0 - _ACP)
_ISA = 1.0 / np.sqrt(_ACP)
_ACP_PREV = np.concatenate([[1.0], _ACP[:-1]])
_C1 = _BETAS * np.sqrt(_ACP_PREV) / (1.0 - _ACP)
_C2 = (1.0 - _ACP_PREV) * np.sqrt(_ALPHAS) / (1.0 - _ACP)
_SIG = np.sqrt(_BETAS * (1.0 - _ACP_PREV) / (1.0 - _ACP))

_half = _TEMB // 2
_freqs = np.exp(-math.log(10000.0) * np.arange(_half, dtype=np.float64) / _half)
_TE = np.concatenate(
    [np.sin(np.arange(_T)[:, None] * _freqs[None, :]),
     np.cos(np.arange(_T)[:, None] * _freqs[None, :])], axis=1
).astype(np.float32)

def _sc_deg_body(edge_hbm, out_hbm, idx_v, ones_v, zb_v, shared_deg, isem):
    cid = lax.axis_index("c")
    sid = lax.axis_index("s")
    wid = cid * 16 + sid

    idescs = [
        pltpu.async_copy(
            edge_hbm.at[1, pl.ds((wid * _EPW_ROWS + j) * _ECHUNK, _ECHUNK)],
            idx_v.at[j], isem)
        for j in range(_EPW_ROWS)
    ]

    def z16(i, carry):
        zb_v[pl.ds(i * 16, 16)] = jnp.zeros((16,), jnp.float32)
        return carry
    lax.fori_loop(0, _SLICE // 16, z16, 0)

    def o16(i, carry):
        ones_v[pl.ds(i * 16, 16)] = jnp.full((16,), 1.0, jnp.float32)
        return carry
    lax.fori_loop(0, _ECHUNK // 16, o16, 0)
    ones_v[pl.ds(_ECHUNK - 16, 16)] = jnp.full((16,), 1.0, jnp.float32)

    pltpu.sync_copy(zb_v, shared_deg.at[pl.ds(sid * _SLICE, _SLICE)])
    for d in idescs:
        d.wait()
    plsc.subcore_barrier()

    def chunk(j, carry):
        pltpu.sync_copy(ones_v, shared_deg.at[idx_v.at[j]], add=True)
        return carry
    lax.fori_loop(0, _EPW_ROWS, chunk, 0)
    plsc.subcore_barrier()
    pltpu.sync_copy(shared_deg.at[pl.ds(sid * _SLICE, _SLICE)],
                    out_hbm.at[cid, pl.ds(sid * _SLICE, _SLICE)])


@functools.lru_cache(maxsize=None)
def _get_deg_call():
    mesh = plsc.VectorSubcoreMesh(
        core_axis_name="c", subcore_axis_name="s", num_cores=2, num_subcores=16)
    return pl.kernel(
        _sc_deg_body,
        out_type=jax.ShapeDtypeStruct((2, _NPAD), jnp.float32),
        mesh=mesh,
        compiler_params=pltpu.CompilerParams(use_tc_tiling_on_sc=False),
        scratch_types=[
            pltpu.VMEM((_EPW_ROWS, _ECHUNK), jnp.int32),
            pltpu.VMEM((_ECHUNK,), jnp.float32),
            pltpu.VMEM((_SLICE,), jnp.float32),
            pltpu.VMEM_SHARED((_NPAD,), jnp.float32),
            pltpu.SemaphoreType.DMA,
        ],
    )


def _sc_msg_body(edge_hbm, proj_hbm, out_hbm,
                 sidx_v, didx_v, rows0, rows1, zb_v, shared_agg,
                 isem, gsem0, gsem1):
    cid = lax.axis_index("c")
    sid = lax.axis_index("s")
    wid = cid * 16 + sid

    idescs = [
        pltpu.async_copy(
            edge_hbm.at[e, pl.ds((wid * _EPW_ROWS + j) * _ECHUNK, _ECHUNK)],
            (sidx_v if e == 0 else didx_v).at[j], isem)
        for e in (0, 1) for j in range(_EPW_ROWS)
    ]

    def z1(i, carry):
        zb_v[i] = jnp.zeros((16,), jnp.float32)
        return carry
    lax.fori_loop(0, _SLICE, z1, 0)

    pltpu.sync_copy(zb_v, shared_agg.at[pl.ds(sid * _SLICE, _SLICE)])
    for d in idescs:
        d.wait()
    plsc.subcore_barrier()

    rows = (rows0, rows1)
    gsems = (gsem0, gsem1)
    descs = [None, None]
    descs[0] = pltpu.async_copy(proj_hbm.at[sidx_v.at[0]], rows0, gsem0)
    for j in range(_EPW_ROWS):
        b = j % 2
        descs[b].wait()
        if j + 1 < _EPW_ROWS:
            nb = (j + 1) % 2
            descs[nb] = pltpu.async_copy(
                proj_hbm.at[sidx_v.at[j + 1]], rows[nb], gsems[nb])
        pltpu.sync_copy(rows[b], shared_agg.at[didx_v.at[j]], add=True)
    plsc.subcore_barrier()
    pltpu.sync_copy(shared_agg.at[pl.ds(sid * _SLICE, _SLICE)],
                    out_hbm.at[cid, pl.ds(sid * _SLICE, _SLICE)])


@functools.lru_cache(maxsize=None)
def _get_msg_call():
    mesh = plsc.VectorSubcoreMesh(
        core_axis_name="c", subcore_axis_name="s", num_cores=2, num_subcores=16)
    return pl.kernel(
        _sc_msg_body,
        out_type=jax.ShapeDtypeStruct((2, _NPAD, _FEAT), jnp.float32),
        mesh=mesh,
        compiler_params=pltpu.CompilerParams(use_tc_tiling_on_sc=False),
        scratch_types=[
            pltpu.VMEM((_EPW_ROWS, _ECHUNK), jnp.int32),
            pltpu.VMEM((_EPW_ROWS, _ECHUNK), jnp.int32),
            pltpu.VMEM((_ECHUNK, _FEAT), jnp.float32),
            pltpu.VMEM((_ECHUNK, _FEAT), jnp.float32),
            pltpu.VMEM((_SLICE, _FEAT), jnp.float32),
            pltpu.VMEM_SHARED((_NPAD, _FEAT), jnp.float32),
            pltpu.SemaphoreType.DMA,
            pltpu.SemaphoreType.DMA,
            pltpu.SemaphoreType.DMA,
        ],
    )


_RB = 2048


def _praw_body(f_ref, wg_ref, out_ref):
    out_ref[...] = jnp.dot(f_ref[...], wg_ref[...],
                           preferred_element_type=jnp.float32)


_praw_call = pl.pallas_call(
    _praw_body,
    in_specs=[
        pl.BlockSpec(memory_space=pltpu.VMEM),
        pl.BlockSpec(memory_space=pltpu.VMEM),
    ],
    out_specs=pl.BlockSpec(memory_space=pltpu.VMEM),
    out_shape=jax.ShapeDtypeStruct((_N, _FEAT), jnp.float32),
)


def _pscale_body(deg_ref, p_ref, out_ref):
    dv = deg_ref[...]
    isd = lax.rsqrt(jnp.maximum(dv[0:1, :_N] + dv[1:2, :_N], 1.0))
    isd_c = jnp.transpose(isd, (1, 0))
    out_ref[...] = p_ref[...] * isd_c


_pscale_call = pl.pallas_call(
    _pscale_body,
    in_specs=[
        pl.BlockSpec(memory_space=pltpu.VMEM),
        pl.BlockSpec(memory_space=pltpu.VMEM),
    ],
    out_specs=pl.BlockSpec(memory_space=pltpu.VMEM),
    out_shape=jax.ShapeDtypeStruct((_N, _FEAT), jnp.float32),
)


def _emb_body(agg_ref, deg_ref, bg_ref, out_ref):
    av = agg_ref[...]
    s = av[0, :_N] + av[1, :_N]
    dv = deg_ref[...]
    isd = lax.rsqrt(jnp.maximum(dv[0:1, :_N] + dv[1:2, :_N], 1.0))
    isd_c = jnp.transpose(isd, (1, 0))
    out_ref[...] = jnp.maximum(s * isd_c + bg_ref[...], 0.0)


_emb_call = pl.pallas_call(
    _emb_body,
    in_specs=[
        pl.BlockSpec(memory_space=pltpu.VMEM),
        pl.BlockSpec(memory_space=pltpu.VMEM),
        pl.BlockSpec(memory_space=pltpu.VMEM),
    ],
    out_specs=pl.BlockSpec(memory_space=pltpu.VMEM),
    out_shape=jax.ShapeDtypeStruct((_N, _FEAT), jnp.float32),
)


_KB = 8000
_NKB = (_N * _FEAT) // _KB
_W1S_OFF = _ACT + _TEMB


def _head_body(state_ref, b1_ref, w2_ref, b2_ref, w3_ref, b3_ref, te_ref,
               xi_ref, nz_ref, w1_hbm, out_ref, wb0, wb1, wbm,
               sem0, sem1, semm):
    cm = pltpu.make_async_copy(w1_hbm.at[pl.ds(0, _W1S_OFF), :], wbm, semm)
    cm.start()
    wbs = (wb0, wb1)
    sems = (sem0, sem1)

    def cp(k, b):
        return pltpu.make_async_copy(
            w1_hbm.at[pl.ds(_W1S_OFF + k * _KB, _KB), :], wbs[b], sems[b])

    cp(0, 0).start()
    acc = jnp.zeros((1, _HID), jnp.float32)
    for k in range(_NKB):
        if k + 1 < _NKB:
            cp(k + 1, (k + 1) % 2).start()
        cp(k, k % 2).wait()
        acc = acc + jnp.dot(state_ref[:, pl.ds(k * _KB, _KB)],
                            wbs[k % 2][...],
                            preferred_element_type=jnp.float32)
    cm.wait()
    wb = wbm
    base = acc + b1_ref[...]
    te_v = te_ref[...]
    x = xi_ref[...]
    w1te = wb[pl.ds(_ACT, _TEMB), :]
    for t in (2, 1, 0):
        tp = jnp.dot(te_v[t:t + 1, :], w1te, preferred_element_type=jnp.float32)
        h = jnp.dot(x, wb[pl.ds(0, _ACT), :], preferred_element_type=jnp.float32)
        h = jnp.maximum(h + tp + base, 0.0)
        h = jnp.maximum(
            jnp.dot(h, w2_ref[...], preferred_element_type=jnp.float32)
            + b2_ref[...], 0.0)
        eps = jnp.dot(h, w3_ref[...], preferred_element_type=jnp.float32) + b3_ref[...]
        x0 = jnp.clip((x - float(_SB[t]) * eps) * float(_ISA[t]),
                      -_MAX_ACTION, _MAX_ACTION)
        x = float(_C1[t]) * x0 + float(_C2[t]) * x
        if t > 0:
            x = x + float(_SIG[t]) * nz_ref[2 - t:3 - t, :]
    out_ref[...] = x


_head_call = pl.pallas_call(
    _head_body,
    in_specs=[
        pl.BlockSpec(memory_space=pltpu.VMEM),
        pl.BlockSpec(memory_space=pltpu.VMEM),
        pl.BlockSpec(memory_space=pltpu.VMEM),
        pl.BlockSpec(memory_space=pltpu.VMEM),
        pl.BlockSpec(memory_space=pltpu.VMEM),
        pl.BlockSpec(memory_space=pltpu.VMEM),
        pl.BlockSpec(memory_space=pltpu.VMEM),
        pl.BlockSpec(memory_space=pltpu.VMEM),
        pl.BlockSpec(memory_space=pltpu.VMEM),
        pl.BlockSpec(memory_space=pl.ANY),
    ],
    out_specs=pl.BlockSpec(memory_space=pltpu.VMEM),
    out_shape=jax.ShapeDtypeStruct((1, _ACT), jnp.float32),
    scratch_shapes=[
        pltpu.VMEM((_KB, _HID), jnp.float32),
        pltpu.VMEM((_KB, _HID), jnp.float32),
        pltpu.VMEM((_W1S_OFF, _HID), jnp.float32),
        pltpu.SemaphoreType.DMA,
        pltpu.SemaphoreType.DMA,
        pltpu.SemaphoreType.DMA,
    ],
)


def kernel(feature_matrix, edge_index, Wg, bg, W1, b1, W2, b2, W3, b3):
    praw = _praw_call(feature_matrix, Wg)
    deg_p = _get_deg_call()(edge_index)
    proj = _pscale_call(deg_p, praw)
    agg_p = _get_msg_call()(edge_index, proj)
    emb = _emb_call(agg_p, deg_p, bg.reshape(1, _FEAT))
    state = emb.reshape(1, _N * _FEAT)

    xi = jax.random.normal(jax.random.key(42), (1, _ACT), dtype=jnp.float32)
    n2 = jax.random.normal(jax.random.fold_in(jax.random.key(7), 2),
                           (1, _ACT), dtype=jnp.float32)
    n1 = jax.random.normal(jax.random.fold_in(jax.random.key(7), 1),
                           (1, _ACT), dtype=jnp.float32)
    noise = jnp.concatenate([n2, n1], axis=0)
    te = jnp.asarray(_TE)

    logits = _head_call(state, b1.reshape(1, _HID), W2, b2.reshape(1, _HID),
                        W3, b3.reshape(1, _ACT), te, xi, noise, W1)
    return logits
